# baseline (device time: 21574 ns/iter reference)
import jax
import jax.numpy as jnp
from jax import lax
from jax.experimental import pallas as pl
from jax.experimental.pallas import tpu as pltpu

N_Z = 4
BLOCK_M = 1024


def kernel(x, dy, gamma):
    del gamma
    m, d = x.shape
    n_steps = m // BLOCK_M

    def body(x_ref, dy_ref, out_ref, acc_ref, comm_ref, send_sems, recv_sems):
        i = pl.program_id(0)

        @pl.when(i == 0)
        def _():
            my_x = lax.axis_index("x")
            my_y = lax.axis_index("y")
            my_z = lax.axis_index("z")
            barrier = pltpu.get_barrier_semaphore()
            for k in range(1, N_Z):
                pl.semaphore_signal(
                    barrier,
                    inc=1,
                    device_id=(my_x, my_y, (my_z + k) % N_Z),
                    device_id_type=pl.DeviceIdType.MESH,
                )
            pl.semaphore_wait(barrier, N_Z - 1)

        xb = x_ref[:, :]
        dyb = dy_ref[:, :]
        mu = jnp.mean(xb, axis=1, keepdims=True)
        xc = xb - mu
        var = jnp.mean(xc * xc, axis=1, keepdims=True)
        xhat = xc * lax.rsqrt(var + 1e-5)
        part = jnp.concatenate(
            [
                jnp.sum(dyb * xhat, axis=0, keepdims=True),
                jnp.sum(dyb, axis=0, keepdims=True),
            ],
            axis=0,
        )

        @pl.when(i == 0)
        def _():
            acc_ref[:, :] = part

        @pl.when(i > 0)
        def _():
            acc_ref[:, :] = acc_ref[:, :] + part

        @pl.when(i == n_steps - 1)
        def _():
            my_x = lax.axis_index("x")
            my_y = lax.axis_index("y")
            my_z = lax.axis_index("z")

            comm_ref[0] = acc_ref[:, :]
            rdmas = []
            for k in range(1, N_Z):
                rdma = pltpu.make_async_remote_copy(
                    src_ref=comm_ref.at[0],
                    dst_ref=comm_ref.at[N_Z - k],
                    send_sem=send_sems.at[k - 1],
                    recv_sem=recv_sems.at[k - 1],
                    device_id=(my_x, my_y, (my_z + k) % N_Z),
                    device_id_type=pl.DeviceIdType.MESH,
                )
                rdma.start()
                rdmas.append(rdma)
            for rdma in rdmas:
                rdma.wait()

            out_ref[:, :] = (comm_ref[0] + comm_ref[1]) + (
                comm_ref[2] + comm_ref[3]
            )

    return pl.pallas_call(
        body,
        grid=(n_steps,),
        in_specs=[
            pl.BlockSpec((BLOCK_M, d), lambda i: (i, 0)),
            pl.BlockSpec((BLOCK_M, d), lambda i: (i, 0)),
        ],
        out_specs=pl.BlockSpec((2, d), lambda i: (0, 0)),
        out_shape=jax.ShapeDtypeStruct((2, d), jnp.float32),
        scratch_shapes=[
            pltpu.VMEM((2, d), jnp.float32),
            pltpu.VMEM((N_Z, 2, d), jnp.float32),
            pltpu.SemaphoreType.DMA((N_Z - 1,)),
            pltpu.SemaphoreType.DMA((N_Z - 1,)),
        ],
        compiler_params=pltpu.CompilerParams(collective_id=0),
    )(x, dy)


# device time: 20777 ns/iter; 1.0384x vs baseline; 1.0384x over previous
import jax
import jax.numpy as jnp
from jax import lax
from jax.experimental import pallas as pl
from jax.experimental.pallas import tpu as pltpu

N_Z = 4
BLOCK_M = 512


def kernel(x, dy, gamma):
    del gamma
    m, d = x.shape
    n_steps = m // BLOCK_M

    def body(x_ref, dy_ref, out_ref, acc_ref, comm_ref, send_sems, recv_sems):
        i = pl.program_id(0)

        @pl.when(i == 0)
        def _():
            my_x = lax.axis_index("x")
            my_y = lax.axis_index("y")
            my_z = lax.axis_index("z")
            barrier = pltpu.get_barrier_semaphore()
            for k in range(1, N_Z):
                pl.semaphore_signal(
                    barrier,
                    inc=1,
                    device_id=(my_x, my_y, (my_z + k) % N_Z),
                    device_id_type=pl.DeviceIdType.MESH,
                )
            pl.semaphore_wait(barrier, N_Z - 1)

        xb = x_ref[:, :]
        dyb = dy_ref[:, :]
        mu = jnp.mean(xb, axis=1, keepdims=True)
        var = jnp.mean(xb * xb, axis=1, keepdims=True) - mu * mu
        a = lax.rsqrt(var + 1e-5)
        b = mu * a
        t = dyb * xb
        w = jnp.concatenate([b, jnp.ones_like(b)], axis=1)
        dn = (((0,), (0,)), ((), ()))
        wdy = lax.dot_general(w, dyb, dn, preferred_element_type=jnp.float32)
        at = lax.dot_general(a, t, dn, preferred_element_type=jnp.float32)
        part = jnp.concatenate([at - wdy[0:1, :], wdy[1:2, :]], axis=0)

        @pl.when(i == 0)
        def _():
            acc_ref[:, :] = part

        @pl.when(i > 0)
        def _():
            acc_ref[:, :] = acc_ref[:, :] + part

        @pl.when(i == n_steps - 1)
        def _():
            my_x = lax.axis_index("x")
            my_y = lax.axis_index("y")
            my_z = lax.axis_index("z")

            comm_ref[0] = acc_ref[:, :]
            rdmas = []
            for k in range(1, N_Z):
                rdma = pltpu.make_async_remote_copy(
                    src_ref=comm_ref.at[0],
                    dst_ref=comm_ref.at[N_Z - k],
                    send_sem=send_sems.at[k - 1],
                    recv_sem=recv_sems.at[k - 1],
                    device_id=(my_x, my_y, (my_z + k) % N_Z),
                    device_id_type=pl.DeviceIdType.MESH,
                )
                rdma.start()
                rdmas.append(rdma)
            for rdma in rdmas:
                rdma.wait()

            out_ref[:, :] = (comm_ref[0] + comm_ref[1]) + (
                comm_ref[2] + comm_ref[3]
            )

    return pl.pallas_call(
        body,
        grid=(n_steps,),
        in_specs=[
            pl.BlockSpec((BLOCK_M, d), lambda i: (i, 0)),
            pl.BlockSpec((BLOCK_M, d), lambda i: (i, 0)),
        ],
        out_specs=pl.BlockSpec((2, d), lambda i: (0, 0)),
        out_shape=jax.ShapeDtypeStruct((2, d), jnp.float32),
        scratch_shapes=[
            pltpu.VMEM((2, d), jnp.float32),
            pltpu.VMEM((N_Z, 2, d), jnp.float32),
            pltpu.SemaphoreType.DMA((N_Z - 1,)),
            pltpu.SemaphoreType.DMA((N_Z - 1,)),
        ],
        compiler_params=pltpu.CompilerParams(collective_id=0),
    )(x, dy)
